# baseline (speedup 1.0000x reference)
import functools

import jax
import jax.numpy as jnp
from jax import lax
from jax.experimental import pallas as pl
from jax.experimental.pallas import tpu as pltpu
from jax.experimental.pallas import tpu_sc as plsc

B = 4096
F = 26
FP = 32
V = 100000
D = 16
R = F * V
G8 = R // 8
R1P = 2600064
G128 = R1P // 128

NC = 2
NS = 16
L = 16
NW = NC * NS
BPW = B // NW
NG = BPW // L
NIDX = FP * BPW


@functools.cache
def _build_fm_sc():
    mesh = plsc.VectorSubcoreMesh(
        core_axis_name="c", subcore_axis_name="s", num_cores=NC, num_subcores=NS
    )

    @functools.partial(
        pl.kernel,
        out_type=jax.ShapeDtypeStruct((B,), jnp.float32),
        mesh=mesh,
        scratch_types=[
            pltpu.VMEM((NIDX,), jnp.int32),
            pltpu.VMEM((NIDX,), jnp.int32),
            pltpu.VMEM((NIDX,), jnp.int32),
            pltpu.VMEM((NIDX,), jnp.int32),
            pltpu.VMEM((BPW, 128), jnp.float32),
            pltpu.VMEM((BPW, 128), jnp.float32),
            pltpu.VMEM((BPW * D,), jnp.float32),
            pltpu.VMEM((BPW * D,), jnp.float32),
            pltpu.VMEM((BPW * D,), jnp.float32),
            pltpu.VMEM((BPW,), jnp.float32),
            pltpu.VMEM((BPW,), jnp.float32),
            pltpu.SemaphoreType.DMA,
            pltpu.SemaphoreType.DMA,
        ],
        compiler_params=pltpu.CompilerParams(
            needs_layout_passes=False, use_tc_tiling_on_sc=True
        ),
    )
    def _fm_sc(idx8_hbm, off2_hbm, idx1_hbm, off1_hbm, emb1_hbm, emb2_hbm,
               out_hbm,
               idx8_v, off2_v, idx1_v, off1_v, buf2_v, buf1_v,
               acc_v, acc2_v, p2_v, p1_v, out_v, sem, sem1):
        w = lax.axis_index("c") * NS + lax.axis_index("s")

        pltpu.sync_copy(idx8_hbm.at[w], idx8_v)
        pltpu.sync_copy(off2_hbm.at[w], off2_v)
        pltpu.sync_copy(idx1_hbm.at[w], idx1_v)
        pltpu.sync_copy(off1_hbm.at[w], off1_v)

        lanes = lax.iota(jnp.int32, L)

        def _fire(f):
            pltpu.make_async_copy(
                emb2_hbm.at[idx8_v.at[pl.ds(f * BPW, BPW)]], buf2_v, sem
            ).start()
            pltpu.make_async_copy(
                emb1_hbm.at[idx1_v.at[pl.ds(f * BPW, BPW)]], buf1_v, sem1
            ).start()

        def _drain(f):
            pltpu.make_async_copy(
                emb2_hbm.at[idx8_v.at[pl.ds(f * BPW, BPW)]], buf2_v, sem
            ).wait()
            pltpu.make_async_copy(
                emb1_hbm.at[idx1_v.at[pl.ds(f * BPW, BPW)]], buf1_v, sem1
            ).wait()

        def _accum(f, first):
            def _sample(s, carry):
                sv = jnp.full((L,), s, jnp.int32)
                ov = plsc.load_gather(off2_v, [jnp.full((L,), f * BPW, jnp.int32) + sv])
                row = plsc.load_gather(buf2_v, [sv, ov + lanes])
                if first:
                    acc_v[pl.ds(s * D, D)] = row
                    acc2_v[pl.ds(s * D, D)] = row * row
                else:
                    acc_v[pl.ds(s * D, D)] = acc_v[pl.ds(s * D, D)] + row
                    acc2_v[pl.ds(s * D, D)] = acc2_v[pl.ds(s * D, D)] + row * row
                return carry

            lax.fori_loop(0, BPW, _sample, 0)

            for g in range(NG):
                o1 = off1_v[pl.ds(f * BPW + g * L, L)]
                vals = plsc.load_gather(buf1_v, [lanes + g * L, o1])
                if first:
                    p1_v[pl.ds(g * L, L)] = vals
                else:
                    p1_v[pl.ds(g * L, L)] = p1_v[pl.ds(g * L, L)] + vals

        for f in range(F):
            _fire(f)
            _drain(f)
            _accum(f, f == 0)

        def _p2(s, carry):
            a = acc_v[pl.ds(s * D, D)]
            a2 = acc2_v[pl.ds(s * D, D)]
            p2_v[pl.ds(s * D, D)] = (a * a - a2) * 0.5
            return carry

        lax.fori_loop(0, BPW, _p2, 0)

        for g in range(NG):
            t = p1_v[pl.ds(g * L, L)]
            sample_base = (lanes + g * L) * D
            for d in range(D):
                t = t + plsc.load_gather(p2_v, [sample_base + d])
            out_v[pl.ds(g * L, L)] = t

        pltpu.sync_copy(out_v, out_hbm.at[pl.ds(w * BPW, BPW)])

    return _fm_sc


def kernel(sparse_idx, emb_first, emb_second):
    flat_idx = sparse_idx + (jnp.arange(F, dtype=sparse_idx.dtype) * V)[None, :]
    per_w = flat_idx.reshape(NW, BPW, F).transpose(0, 2, 1)
    per_w = jnp.pad(per_w, ((0, 0), (0, FP - F), (0, 0)))
    s_off = jnp.arange(BPW, dtype=jnp.int32)[None, None, :]

    idx8 = (per_w // 8).reshape(NW, NIDX)
    off2 = ((per_w % 8) * D).reshape(NW, NIDX)
    idx1 = (per_w // 128).reshape(NW, NIDX)
    off1 = (per_w % 128).reshape(NW, NIDX)

    emb2_rec = emb_second.reshape(G8, 8 * D)
    emb1_rec = jnp.pad(emb_first.reshape(-1), (0, R1P - R)).reshape(G128, 128)

    out = _build_fm_sc()(idx8, off2, idx1, off1, emb1_rec, emb2_rec)
    return out[:, None]
